# baseline (speedup 1.0000x reference)
import functools

import jax
import jax.numpy as jnp
from jax import lax
from jax.experimental import pallas as pl
from jax.experimental.pallas import tpu as pltpu
from jax.experimental.pallas import tpu_sc as plsc

N_NODES = 2048
E = 32768
IN = 32
OUT = 128
EF = 16
ZDIM = EF * IN

NC, NS = 2, 16
NW = NC * NS
EPW = E // NW
C = 128
K = EPW // C

def _gather_body(n_hbm, src_hbm, h_out, idx_v, rows_v, sem0, sem1):
    cc = lax.axis_index("c")
    ss = lax.axis_index("s")
    wid = ss * NC + cc
    sems = (sem0, sem1)
    pltpu.sync_copy(src_hbm.at[wid], idx_v)
    prev = pltpu.async_copy(n_hbm.at[idx_v.at[0]], rows_v.at[0], sems[0])
    for k in range(1, K):
        cur = pltpu.async_copy(n_hbm.at[idx_v.at[k]], rows_v.at[k % 2], sems[k % 2])
        prev.wait()
        pltpu.sync_copy(
            rows_v.at[(k - 1) % 2], h_out.at[pl.ds(wid * EPW + (k - 1) * C, C)]
        )
        prev = cur
    prev.wait()
    pltpu.sync_copy(
        rows_v.at[(K - 1) % 2], h_out.at[pl.ds(wid * EPW + (K - 1) * C, C)]
    )


def _msg_body(e_ref, h_ref, r_ref, t_ref, w2_ref, bm_ref, m_ref):
    e_blk = e_ref[...]
    h_blk = h_ref[...]
    e_exp = lax.dot_general(
        e_blk, r_ref[...], (((1,), (0,)), ((), ())),
        preferred_element_type=jnp.float32,
    )
    h_til = lax.dot_general(
        h_blk, t_ref[...], (((1,), (0,)), ((), ())),
        preferred_element_type=jnp.float32,
    )
    z = e_exp * h_til
    acc = lax.dot_general(
        z, w2_ref[...], (((1,), (0,)), ((), ())), preferred_element_type=jnp.float32
    )
    acc = acc + lax.dot_general(
        h_blk, bm_ref[...], (((1,), (0,)), ((), ())),
        preferred_element_type=jnp.float32,
    )
    m_ref[...] = acc


def _msg_call(e, h_src, rmat, tmat, w2, bmat):
    blk = 2048
    grid = E // blk
    return pl.pallas_call(
        _msg_body,
        grid=(grid,),
        in_specs=[
            pl.BlockSpec((blk, EF), lambda i: (i, 0)),
            pl.BlockSpec((blk, OUT), lambda i: (i, 0)),
            pl.BlockSpec((EF, ZDIM), lambda i: (0, 0)),
            pl.BlockSpec((OUT, ZDIM), lambda i: (0, 0)),
            pl.BlockSpec((ZDIM, OUT), lambda i: (0, 0)),
            pl.BlockSpec((OUT, OUT), lambda i: (0, 0)),
        ],
        out_specs=pl.BlockSpec((blk, OUT), lambda i: (i, 0)),
        out_shape=jax.ShapeDtypeStruct((E, OUT), jnp.float32),
    )(e, h_src, rmat, tmat, w2, bmat)


_RPS = N_NODES // NS


def _scatter_body(m_hbm, dst_hbm, zeros_hbm, out_hbm, idx_v, m_v, agg_sh):
    cc = lax.axis_index("c")
    ss = lax.axis_index("s")
    wid = ss * NC + cc
    pltpu.sync_copy(
        zeros_hbm.at[pl.ds(ss * _RPS, _RPS)], agg_sh.at[pl.ds(ss * _RPS, _RPS)]
    )
    pltpu.sync_copy(dst_hbm.at[wid], idx_v)
    plsc.subcore_barrier()
    for k in range(K):
        pltpu.sync_copy(m_hbm.at[pl.ds(wid * EPW + k * C, C)], m_v)
        pltpu.sync_copy(m_v, agg_sh.at[idx_v.at[k]], add=True)
    plsc.subcore_barrier()
    pltpu.sync_copy(
        agg_sh.at[pl.ds(ss * _RPS, _RPS)], out_hbm.at[cc, pl.ds(ss * _RPS, _RPS)]
    )


@functools.lru_cache(maxsize=None)
def _sc_kernels():
    mesh = plsc.VectorSubcoreMesh(
        core_axis_name="c", subcore_axis_name="s", num_cores=NC, num_subcores=NS
    )
    gather = pl.kernel(
        _gather_body,
        mesh=mesh,
        out_type=jax.ShapeDtypeStruct((E, OUT), jnp.float32),
        scratch_types=[
            pltpu.VMEM((K, C), jnp.int32),
            pltpu.VMEM((2, C, OUT), jnp.float32),
            pltpu.SemaphoreType.DMA,
            pltpu.SemaphoreType.DMA,
        ],
    )
    scatter = pl.kernel(
        _scatter_body,
        mesh=mesh,
        out_type=jax.ShapeDtypeStruct((NC, N_NODES, OUT), jnp.float32),
        scratch_types=[
            pltpu.VMEM((K, C), jnp.int32),
            pltpu.VMEM((C, OUT), jnp.float32),
            pltpu.VMEM_SHARED((N_NODES, OUT), jnp.float32),
        ],
    )
    return gather, scatter


def _pool_body(p_ref, cb_ref, gw_ref, gb_ref, f1w_ref, f1b_ref, f2w_ref, f2b_ref,
               o_ref):
    h = p_ref[0] + p_ref[1] + cb_ref[...]
    g = jnp.sum(h * gw_ref[...], axis=1, keepdims=True) + gb_ref[...]
    gmax = jnp.max(g)
    ex = jnp.exp(g - gmax)
    gate = ex / jnp.sum(ex)
    readout = jnp.sum(gate * h, axis=0, keepdims=True)
    h2 = jnp.where(readout > 0, readout, jnp.exp(readout) - 1.0)
    t = lax.dot_general(
        h2, f1w_ref[...], (((1,), (0,)), ((), ())), preferred_element_type=jnp.float32
    )
    t = jnp.maximum(t + f1b_ref[...], 0.0)
    o = lax.dot_general(
        t, f2w_ref[...], (((1,), (0,)), ((), ())), preferred_element_type=jnp.float32
    )
    o_ref[...] = o + f2b_ref[...]


def _pool_call(partials, conv_bias, gate_w, gate_b, f1w, f1b, f2w, f2b):
    return pl.pallas_call(
        _pool_body,
        out_shape=jax.ShapeDtypeStruct((1, 1), jnp.float32),
    )(partials, conv_bias, gate_w, gate_b, f1w, f1b, f2w, f2b)


def kernel(n, edge_index, e, W_edge, b_edge, conv_bias, gate_W, gate_b,
           fc1_W, fc1_b, fc2_W, fc2_b):
    src = edge_index[0].reshape(NW, K, C)
    dst = edge_index[1].reshape(NW, K, C)
    w2 = W_edge.reshape(ZDIM, OUT)
    bmat = jnp.zeros((OUT, OUT), jnp.float32).at[:IN].set(b_edge.reshape(IN, OUT))
    zeros = jnp.zeros((N_NODES, OUT), jnp.float32)
    n_pad = jnp.zeros((N_NODES, OUT), jnp.float32).at[:, :IN].set(n)
    lane = jnp.arange(ZDIM, dtype=jnp.int32)
    rmat = (lane[None, :] // IN == jnp.arange(EF, dtype=jnp.int32)[:, None]
            ).astype(jnp.float32)
    tmat = (lane[None, :] % IN == jnp.arange(OUT, dtype=jnp.int32)[:, None]
            ).astype(jnp.float32)

    gather_k, scatter_k = _sc_kernels()
    h_src = gather_k(n_pad, src)
    m = _msg_call(e, h_src, rmat, tmat, w2, bmat)
    partials = scatter_k(m, dst, zeros)

    out = _pool_call(
        partials,
        conv_bias.reshape(1, OUT),
        gate_W.reshape(1, OUT),
        gate_b.reshape(1, 1),
        fc1_W,
        fc1_b.reshape(1, 32),
        fc2_W,
        fc2_b.reshape(1, 1),
    )
    return out


# speedup vs baseline: 1.0735x; 1.0735x over previous
import functools

import jax
import jax.numpy as jnp
from jax import lax
from jax.experimental import pallas as pl
from jax.experimental.pallas import tpu as pltpu
from jax.experimental.pallas import tpu_sc as plsc

N_NODES = 2048
E = 32768
IN = 32
OUT = 128
EF = 16
ZDIM = EF * IN

NC, NS = 2, 16
NW = NC * NS
EPW = E // NW
C = 128
K = EPW // C

def _gather_body(n_hbm, src_hbm, h_out, idx_v, rows_v, sem0, sem1):
    cc = lax.axis_index("c")
    ss = lax.axis_index("s")
    wid = ss * NC + cc
    sems = (sem0, sem1)
    pltpu.sync_copy(src_hbm.at[wid], idx_v)
    prev = pltpu.async_copy(n_hbm.at[idx_v.at[0]], rows_v.at[0], sems[0])
    for k in range(1, K):
        cur = pltpu.async_copy(n_hbm.at[idx_v.at[k]], rows_v.at[k % 2], sems[k % 2])
        prev.wait()
        pltpu.sync_copy(
            rows_v.at[(k - 1) % 2], h_out.at[pl.ds(wid * EPW + (k - 1) * C, C)]
        )
        prev = cur
    prev.wait()
    pltpu.sync_copy(
        rows_v.at[(K - 1) % 2], h_out.at[pl.ds(wid * EPW + (K - 1) * C, C)]
    )


def _msg_body(e_ref, h_ref, w2_ref, bm_ref, m_ref):
    e_blk = e_ref[...]
    h_blk = h_ref[...]
    lane_r = lax.broadcasted_iota(jnp.int32, (EF, ZDIM), 1)
    row_r = lax.broadcasted_iota(jnp.int32, (EF, ZDIM), 0)
    rmat = (lane_r // IN == row_r).astype(jnp.float32)
    lane_t = lax.broadcasted_iota(jnp.int32, (IN, ZDIM), 1)
    row_t = lax.broadcasted_iota(jnp.int32, (IN, ZDIM), 0)
    tmat = (lane_t % IN == row_t).astype(jnp.float32)
    e_exp = lax.dot_general(
        e_blk, rmat, (((1,), (0,)), ((), ())),
        preferred_element_type=jnp.float32,
    )
    h_til = lax.dot_general(
        h_blk, tmat, (((1,), (0,)), ((), ())),
        preferred_element_type=jnp.float32,
    )
    z = e_exp * h_til
    acc = lax.dot_general(
        z, w2_ref[...], (((1,), (0,)), ((), ())), preferred_element_type=jnp.float32
    )
    acc = acc + lax.dot_general(
        h_blk, bm_ref[...], (((1,), (0,)), ((), ())),
        preferred_element_type=jnp.float32,
    )
    m_ref[...] = acc


def _msg_call(e, h_src, w2, bmat):
    blk = 2048
    grid = E // blk
    return pl.pallas_call(
        _msg_body,
        grid=(grid,),
        in_specs=[
            pl.BlockSpec((blk, EF), lambda i: (i, 0)),
            pl.BlockSpec((blk, IN), lambda i: (i, 0)),
            pl.BlockSpec((ZDIM, OUT), lambda i: (0, 0)),
            pl.BlockSpec((IN, OUT), lambda i: (0, 0)),
        ],
        out_specs=pl.BlockSpec((blk, OUT), lambda i: (i, 0)),
        out_shape=jax.ShapeDtypeStruct((E, OUT), jnp.float32),
    )(e, h_src, w2, bmat)


_RPS = N_NODES // NS


def _scatter_body(m_hbm, dst_hbm, zeros_hbm, out_hbm, idx_v, m_v, agg_sh,
                  sem0, sem1):
    cc = lax.axis_index("c")
    ss = lax.axis_index("s")
    wid = ss * NC + cc
    sems = (sem0, sem1)
    prev = pltpu.async_copy(m_hbm.at[pl.ds(wid * EPW, C)], m_v.at[0], sems[0])
    pltpu.sync_copy(
        zeros_hbm.at[pl.ds(ss * _RPS, _RPS)], agg_sh.at[pl.ds(ss * _RPS, _RPS)]
    )
    pltpu.sync_copy(dst_hbm.at[wid], idx_v)
    plsc.subcore_barrier()
    for k in range(K):
        if k + 1 < K:
            nxt = pltpu.async_copy(
                m_hbm.at[pl.ds(wid * EPW + (k + 1) * C, C)],
                m_v.at[(k + 1) % 2],
                sems[(k + 1) % 2],
            )
        prev.wait()
        pltpu.sync_copy(m_v.at[k % 2], agg_sh.at[idx_v.at[k]], add=True)
        if k + 1 < K:
            prev = nxt
    plsc.subcore_barrier()
    pltpu.sync_copy(
        agg_sh.at[pl.ds(ss * _RPS, _RPS)], out_hbm.at[cc, pl.ds(ss * _RPS, _RPS)]
    )


@functools.lru_cache(maxsize=None)
def _sc_kernels():
    mesh = plsc.VectorSubcoreMesh(
        core_axis_name="c", subcore_axis_name="s", num_cores=NC, num_subcores=NS
    )
    gather = pl.kernel(
        _gather_body,
        mesh=mesh,
        out_type=jax.ShapeDtypeStruct((E, IN), jnp.float32),
        scratch_types=[
            pltpu.VMEM((K, C), jnp.int32),
            pltpu.VMEM((2, C, IN), jnp.float32),
            pltpu.SemaphoreType.DMA,
            pltpu.SemaphoreType.DMA,
        ],
        compiler_params=pltpu.CompilerParams(use_tc_tiling_on_sc=False),
    )
    scatter = pl.kernel(
        _scatter_body,
        mesh=mesh,
        out_type=jax.ShapeDtypeStruct((NC, N_NODES, OUT), jnp.float32),
        scratch_types=[
            pltpu.VMEM((K, C), jnp.int32),
            pltpu.VMEM((2, C, OUT), jnp.float32),
            pltpu.VMEM_SHARED((N_NODES, OUT), jnp.float32),
            pltpu.SemaphoreType.DMA,
            pltpu.SemaphoreType.DMA,
        ],
    )
    return gather, scatter


def _pool_body(p_ref, cb_ref, gw_ref, gb_ref, f1w_ref, f1b_ref, f2w_ref, f2b_ref,
               o_ref):
    h = p_ref[0] + p_ref[1] + cb_ref[...]
    g = jnp.sum(h * gw_ref[...], axis=1, keepdims=True) + gb_ref[...]
    gmax = jnp.max(g)
    ex = jnp.exp(g - gmax)
    gate = ex / jnp.sum(ex)
    readout = jnp.sum(gate * h, axis=0, keepdims=True)
    h2 = jnp.where(readout > 0, readout, jnp.exp(readout) - 1.0)
    t = lax.dot_general(
        h2, f1w_ref[...], (((1,), (0,)), ((), ())), preferred_element_type=jnp.float32
    )
    t = jnp.maximum(t + f1b_ref[...], 0.0)
    o = lax.dot_general(
        t, f2w_ref[...], (((1,), (0,)), ((), ())), preferred_element_type=jnp.float32
    )
    o_ref[...] = o + f2b_ref[...]


def _pool_call(partials, conv_bias, gate_w, gate_b, f1w, f1b, f2w, f2b):
    return pl.pallas_call(
        _pool_body,
        out_shape=jax.ShapeDtypeStruct((1, 1), jnp.float32),
    )(partials, conv_bias, gate_w, gate_b, f1w, f1b, f2w, f2b)


def kernel(n, edge_index, e, W_edge, b_edge, conv_bias, gate_W, gate_b,
           fc1_W, fc1_b, fc2_W, fc2_b):
    src = edge_index[0].reshape(NW, K, C)
    dst = edge_index[1].reshape(NW, K, C)
    w2 = W_edge.reshape(ZDIM, OUT)
    bmat = b_edge.reshape(IN, OUT)
    zeros = jnp.zeros((N_NODES, OUT), jnp.float32)

    gather_k, scatter_k = _sc_kernels()
    h_src = gather_k(n, src)
    m = _msg_call(e, h_src, w2, bmat)
    partials = scatter_k(m, dst, zeros)

    out = _pool_call(
        partials,
        conv_bias.reshape(1, OUT),
        gate_W.reshape(1, OUT),
        gate_b.reshape(1, 1),
        fc1_W,
        fc1_b.reshape(1, 32),
        fc2_W,
        fc2_b.reshape(1, 1),
    )
    return out
